# baseline (speedup 1.0000x reference)
import jax
import jax.numpy as jnp
from jax import lax
from jax.experimental import pallas as pl
from jax.experimental.pallas import tpu as pltpu
from jax.experimental.pallas import tpu_sc as plsc

N = 10000
E = 320000
D = 128
DH = D // 2

NC = 2
NS = 16
C = 128
EP = 327680
CH = EP // (NS * C)
ACC_ROWS = 10240
ZR = 64
DW = 16
ROWS_Z = ACC_ROWS // NS
K_DEG = 8
K_NODEG = 8

_mesh = plsc.VectorSubcoreMesh(
    core_axis_name="c", subcore_axis_name="s", num_cores=NC, num_subcores=NS)


def _fill(ref, rows, width, value):
  lanes = 32 if ref.dtype == jnp.bfloat16 else 16
  v = jnp.full((lanes,), value, ref.dtype)
  for r in range(rows):
    for j in range(width // lanes):
      ref[r, pl.ds(j * lanes, lanes)] = v


def _sc_agg_build(with_deg):
  K = K_DEG if with_deg else K_NODEG

  def body(tbl_hbm, srcs_hbm, dsts_hbm, *rest):
    if with_deg:
      (out_hbm, deg_hbm, src_idx, dst_idx, rows, zrow, dzrow, ones, acc,
       dacc, sem, ssem) = rest
    else:
      (out_hbm, src_idx, dst_idx, rows, zrow, acc, sem, ssem) = rest
    cid = lax.axis_index("c")
    sid = lax.axis_index("s")

    _fill(zrow, ZR, DH, 0.0)
    if with_deg:
      _fill(dzrow, ZR, DW, 0.0)
      _fill(ones, C, DW, 1.0)

    zbase = sid * ROWS_Z
    cps = []
    for k in range(ROWS_Z // ZR):
      cps.append(pltpu.async_copy(zrow, acc.at[pl.ds(zbase + k * ZR, ZR)], sem))
      if with_deg:
        cps.append(
            pltpu.async_copy(dzrow, dacc.at[pl.ds(zbase + k * ZR, ZR)], sem))
    pltpu.sync_copy(srcs_hbm.at[cid, sid], src_idx)
    pltpu.sync_copy(dsts_hbm.at[sid], dst_idx)
    for cp in cps:
      cp.wait()
    plsc.subcore_barrier()

    def prime(b, carry):
      pltpu.async_copy(tbl_hbm.at[src_idx.at[b]], rows.at[b], sem)
      return carry

    lax.fori_loop(0, K, prime, 0)

    def chunk(i, carry):
      b = lax.rem(i, K)
      pltpu.make_async_copy(tbl_hbm.at[src_idx.at[i]], rows.at[b],
                            sem).wait()
      pltpu.async_copy(rows.at[b], acc.at[dst_idx.at[i]], ssem, add=True)
      if with_deg:
        @pl.when(cid == 0)
        def _():
          pltpu.async_copy(ones, dacc.at[dst_idx.at[i]], ssem, add=True)
      pltpu.make_async_copy(rows.at[b], acc.at[dst_idx.at[i]], ssem).wait()
      if with_deg:
        @pl.when(cid == 0)
        def _():
          pltpu.make_async_copy(ones, dacc.at[dst_idx.at[i]], ssem).wait()
      nxt = i + K

      @pl.when(nxt < CH)
      def _():
        pltpu.async_copy(tbl_hbm.at[src_idx.at[nxt]], rows.at[b], sem)
      return carry

    lax.fori_loop(0, CH, chunk, 0)
    plsc.subcore_barrier()

    pltpu.sync_copy(acc.at[pl.ds(zbase, ROWS_Z)],
                    out_hbm.at[cid, pl.ds(zbase, ROWS_Z)])
    if with_deg:
      @pl.when(cid == 0)
      def _():
        pltpu.sync_copy(dacc.at[pl.ds(zbase, ROWS_Z)],
                        deg_hbm.at[pl.ds(zbase, ROWS_Z)])

  out_type = [jax.ShapeDtypeStruct((NC, ACC_ROWS, DH), jnp.bfloat16)]
  scratch = [
      pltpu.VMEM((CH, C), jnp.int32),
      pltpu.VMEM((CH, C), jnp.int32),
      pltpu.VMEM((K, C, DH), jnp.bfloat16),
      pltpu.VMEM((ZR, DH), jnp.bfloat16),
  ]
  if with_deg:
    out_type.append(jax.ShapeDtypeStruct((ACC_ROWS, DW), jnp.float32))
    scratch += [
        pltpu.VMEM((ZR, DW), jnp.float32),
        pltpu.VMEM((C, DW), jnp.float32),
    ]
  scratch.append(pltpu.VMEM_SHARED((ACC_ROWS, DH), jnp.bfloat16))
  if with_deg:
    scratch.append(pltpu.VMEM_SHARED((ACC_ROWS, DW), jnp.float32))
  scratch.append(pltpu.SemaphoreType.DMA)
  scratch.append(pltpu.SemaphoreType.DMA)

  return pl.kernel(body, out_type=tuple(out_type), mesh=_mesh,
                   scratch_types=tuple(scratch),
                   compiler_params=pltpu.CompilerParams(
                       use_tc_tiling_on_sc=False))


_sc_agg_deg = _sc_agg_build(True)
_sc_agg = _sc_agg_build(False)

BR = 1000
GRID = N // BR


def _neigh(p, dp):
  n = jnp.concatenate([p[0], p[1]], axis=1).astype(jnp.float32)
  scale = 1.0 / jnp.maximum(dp[:, 0:1], 1.0)
  return n * scale


def _layer(h, n, ws, wn, b):
  return jnp.maximum(
      jnp.dot(h, ws, preferred_element_type=jnp.float32)
      + jnp.dot(n, wn, preferred_element_type=jnp.float32) + b, 0.0)


def _tc_layer_body(x_ref, p_ref, dp_ref, ws_ref, wn_ref, b_ref, o_ref,
                   ob_ref):
  h1 = _layer(x_ref[...], _neigh(p_ref[...], dp_ref[...]), ws_ref[...],
              wn_ref[...], b_ref[...])
  o_ref[...] = h1
  ob_ref[...] = h1.astype(jnp.bfloat16)


def _tc_layer_pool_body(h_ref, p_ref, dp_ref, ws_ref, wn_ref, b_ref, o_ref):
  i = pl.program_id(0)
  h2 = _layer(h_ref[...], _neigh(p_ref[...], dp_ref[...]), ws_ref[...],
              wn_ref[...], b_ref[...])

  @pl.when(i == 0)
  def _():
    o_ref[...] = jnp.zeros_like(o_ref)

  o_ref[...] += jnp.sum(h2, axis=0, keepdims=True)


def _tc_head_body(hs_ref, w1_ref, b1_ref, w2t_ref, b2_ref, o_ref):
  hg = hs_ref[...] * (1.0 / N)
  a = jnp.maximum(
      jnp.dot(hg, w1_ref[...], preferred_element_type=jnp.float32)
      + b1_ref[...], 0.0)
  o = jnp.sum(a * w2t_ref[...], axis=1, keepdims=True) + b2_ref[...]
  o_ref[...] = jax.nn.sigmoid(o)


_row_spec = pl.BlockSpec((BR, D), lambda i: (i, 0))
_split_spec = pl.BlockSpec((NC, BR, DH), lambda i: (0, i, 0))
_dp_spec = pl.BlockSpec((BR, DW), lambda i: (i, 0))
_w_spec = pl.BlockSpec((D, D), lambda i: (0, 0))
_b_spec = pl.BlockSpec((1, D), lambda i: (0, 0))

_tc_layer = pl.pallas_call(
    _tc_layer_body,
    grid=(GRID,),
    in_specs=[_row_spec, _split_spec, _dp_spec, _w_spec, _w_spec, _b_spec],
    out_specs=[_row_spec, pl.BlockSpec((BR, D), lambda i: (i, 0))],
    out_shape=[jax.ShapeDtypeStruct((N, D), jnp.float32),
               jax.ShapeDtypeStruct((N, D), jnp.bfloat16)],
)

_tc_layer_pool = pl.pallas_call(
    _tc_layer_pool_body,
    grid=(GRID,),
    in_specs=[_row_spec, _split_spec, _dp_spec, _w_spec, _w_spec, _b_spec],
    out_specs=pl.BlockSpec((1, D), lambda i: (0, 0)),
    out_shape=jax.ShapeDtypeStruct((1, D), jnp.float32),
)

_tc_head = pl.pallas_call(
    _tc_head_body,
    in_specs=[pl.BlockSpec((1, D), lambda: (0, 0)),
              pl.BlockSpec((D, D), lambda: (0, 0)),
              pl.BlockSpec((1, D), lambda: (0, 0)),
              pl.BlockSpec((1, D), lambda: (0, 0)),
              pl.BlockSpec((1, 1), lambda: (0, 0))],
    out_specs=pl.BlockSpec((1, 1), lambda: (0, 0)),
    out_shape=jax.ShapeDtypeStruct((1, 1), jnp.float32),
)


def kernel(x, edge_index, W_self1, W_neigh1, b1, W_self2, W_neigh2, b2,
           fc1_W, fc1_b, fc2_W, fc2_b):
  pad = EP - E
  srcp = jnp.concatenate([edge_index[0], jnp.zeros((pad,), jnp.int32)])
  dstp = jnp.concatenate([edge_index[1], jnp.full((pad,), N, jnp.int32)])
  srcs = jnp.stack([2 * srcp, 2 * srcp + 1]).reshape(NC, NS, CH, C)
  dsts = dstp.reshape(NS, CH, C)

  p1, deg = _sc_agg_deg(x.astype(jnp.bfloat16).reshape(2 * N, DH), srcs, dsts)
  h1, h1b = _tc_layer(x, p1, deg, W_self1, W_neigh1, b1.reshape(1, D))
  (p2,) = _sc_agg(h1b.reshape(2 * N, DH), srcs, dsts)
  hsum = _tc_layer_pool(h1, p2, deg, W_self2, W_neigh2, b2.reshape(1, D))
  return _tc_head(hsum, fc1_W, fc1_b.reshape(1, D), fc2_W.reshape(1, D),
                  fc2_b.reshape(1, 1))


# speedup vs baseline: 1.0158x; 1.0158x over previous
import jax
import jax.numpy as jnp
from jax import lax
from jax.experimental import pallas as pl
from jax.experimental.pallas import tpu as pltpu
from jax.experimental.pallas import tpu_sc as plsc

N = 10000
E = 320000
D = 128
DH = D // 2

NC = 2
NS = 16
C = 128
EP = 327680
CH = EP // (NS * C)
ACC_ROWS = 10240
ZR = 64
DW = 16
ROWS_Z = ACC_ROWS // NS
K_DEG = 8
K_NODEG = 8

_mesh = plsc.VectorSubcoreMesh(
    core_axis_name="c", subcore_axis_name="s", num_cores=NC, num_subcores=NS)


def _fill(ref, rows, width, value):
  lanes = 32 if ref.dtype == jnp.bfloat16 else 16
  v = jnp.full((lanes,), value, ref.dtype)
  for r in range(rows):
    for j in range(width // lanes):
      ref[r, pl.ds(j * lanes, lanes)] = v


def _sc_agg_build(with_deg):
  K = K_DEG if with_deg else K_NODEG

  def body(tbl_hbm, srcs_hbm, dsts_hbm, *rest):
    if with_deg:
      (out_hbm, deg_hbm, src_idx, dst_idx, rows, zrow, dzrow, ones, acc,
       dacc, sem, ssem) = rest
    else:
      (out_hbm, src_idx, dst_idx, rows, zrow, acc, sem, ssem) = rest
    cid = lax.axis_index("c")
    sid = lax.axis_index("s")

    _fill(zrow, ZR, DH, 0.0)
    if with_deg:
      _fill(dzrow, ZR, DW, 0.0)
      _fill(ones, C, DW, 1.0)

    zbase = sid * ROWS_Z
    cps = []
    for k in range(ROWS_Z // ZR):
      cps.append(pltpu.async_copy(zrow, acc.at[pl.ds(zbase + k * ZR, ZR)], sem))
      if with_deg:
        cps.append(
            pltpu.async_copy(dzrow, dacc.at[pl.ds(zbase + k * ZR, ZR)], sem))
    pltpu.sync_copy(srcs_hbm.at[cid, sid], src_idx)
    pltpu.sync_copy(dsts_hbm.at[sid], dst_idx)
    for cp in cps:
      cp.wait()
    plsc.subcore_barrier()

    def prime(b, carry):
      pltpu.async_copy(tbl_hbm.at[src_idx.at[b]], rows.at[b], sem)
      return carry

    lax.fori_loop(0, K, prime, 0)

    def chunk(i, carry):
      b = lax.rem(i, K)
      pltpu.make_async_copy(tbl_hbm.at[src_idx.at[i]], rows.at[b],
                            sem).wait()
      pltpu.async_copy(rows.at[b], acc.at[dst_idx.at[i]], ssem, add=True)
      if with_deg:
        @pl.when(cid == 0)
        def _():
          pltpu.async_copy(ones, dacc.at[dst_idx.at[i]], ssem, add=True)
      pltpu.make_async_copy(rows.at[b], acc.at[dst_idx.at[i]], ssem).wait()
      if with_deg:
        @pl.when(cid == 0)
        def _():
          pltpu.make_async_copy(ones, dacc.at[dst_idx.at[i]], ssem).wait()
      nxt = i + K

      @pl.when(nxt < CH)
      def _():
        pltpu.async_copy(tbl_hbm.at[src_idx.at[nxt]], rows.at[b], sem)
      return carry

    lax.fori_loop(0, CH, chunk, 0)
    plsc.subcore_barrier()

    pltpu.sync_copy(acc.at[pl.ds(zbase, ROWS_Z)],
                    out_hbm.at[cid, pl.ds(zbase, ROWS_Z)])
    if with_deg:
      @pl.when(cid == 0)
      def _():
        pltpu.sync_copy(dacc.at[pl.ds(zbase, ROWS_Z)],
                        deg_hbm.at[pl.ds(zbase, ROWS_Z)])

  out_type = [jax.ShapeDtypeStruct((NC, ACC_ROWS, DH), jnp.bfloat16)]
  scratch = [
      pltpu.VMEM((CH, C), jnp.int32),
      pltpu.VMEM((CH, C), jnp.int32),
      pltpu.VMEM((K, C, DH), jnp.bfloat16),
      pltpu.VMEM((ZR, DH), jnp.bfloat16),
  ]
  if with_deg:
    out_type.append(jax.ShapeDtypeStruct((ACC_ROWS, DW), jnp.float32))
    scratch += [
        pltpu.VMEM((ZR, DW), jnp.float32),
        pltpu.VMEM((C, DW), jnp.float32),
    ]
  scratch.append(pltpu.VMEM_SHARED((ACC_ROWS, DH), jnp.bfloat16))
  if with_deg:
    scratch.append(pltpu.VMEM_SHARED((ACC_ROWS, DW), jnp.float32))
  scratch.append(pltpu.SemaphoreType.DMA)
  scratch.append(pltpu.SemaphoreType.DMA)

  return pl.kernel(body, out_type=tuple(out_type), mesh=_mesh,
                   scratch_types=tuple(scratch),
                   compiler_params=pltpu.CompilerParams(
                       use_tc_tiling_on_sc=False))


_sc_agg_deg = _sc_agg_build(True)
_sc_agg = _sc_agg_build(False)

BR = 1000
GRID = N // BR


def _neigh(p, dp):
  n = jnp.concatenate([p[0], p[1]], axis=1).astype(jnp.float32)
  scale = 1.0 / jnp.maximum(dp[:, 0:1], 1.0)
  return n * scale


def _layer(h, n, ws, wn, b):
  return jnp.maximum(
      jnp.dot(h, ws, preferred_element_type=jnp.float32)
      + jnp.dot(n, wn, preferred_element_type=jnp.float32) + b, 0.0)


def _tc_layer_body(x_ref, p_ref, dp_ref, ws_ref, wn_ref, b_ref, ob_ref):
  h1 = _layer(x_ref[...], _neigh(p_ref[...], dp_ref[...]), ws_ref[...],
              wn_ref[...], b_ref[...])
  ob_ref[...] = h1.astype(jnp.bfloat16)


def _tc_layer_pool_body(h_ref, p_ref, dp_ref, ws_ref, wn_ref, b_ref,
                        w1_ref, b1_ref, w2t_ref, b2_ref, o_ref, acc_ref):
  i = pl.program_id(0)
  h2 = _layer(h_ref[...].astype(jnp.float32),
              _neigh(p_ref[...], dp_ref[...]), ws_ref[...], wn_ref[...],
              b_ref[...])

  @pl.when(i == 0)
  def _():
    acc_ref[...] = jnp.zeros_like(acc_ref)

  acc_ref[...] += jnp.sum(h2, axis=0, keepdims=True)

  @pl.when(i == GRID - 1)
  def _():
    hg = acc_ref[...] * (1.0 / N)
    a = jnp.maximum(
        jnp.dot(hg, w1_ref[...], preferred_element_type=jnp.float32)
        + b1_ref[...], 0.0)
    o = jnp.sum(a * w2t_ref[...], axis=1, keepdims=True) + b2_ref[...]
    o_ref[...] = jax.nn.sigmoid(o)


_row_spec = pl.BlockSpec((BR, D), lambda i: (i, 0))
_split_spec = pl.BlockSpec((NC, BR, DH), lambda i: (0, i, 0))
_dp_spec = pl.BlockSpec((BR, DW), lambda i: (i, 0))
_w_spec = pl.BlockSpec((D, D), lambda i: (0, 0))
_b_spec = pl.BlockSpec((1, D), lambda i: (0, 0))

_tc_layer = pl.pallas_call(
    _tc_layer_body,
    grid=(GRID,),
    in_specs=[_row_spec, _split_spec, _dp_spec, _w_spec, _w_spec, _b_spec],
    out_specs=_row_spec,
    out_shape=jax.ShapeDtypeStruct((N, D), jnp.bfloat16),
)

_tc_layer_pool = pl.pallas_call(
    _tc_layer_pool_body,
    grid=(GRID,),
    in_specs=[_row_spec, _split_spec, _dp_spec, _w_spec, _w_spec, _b_spec,
              _w_spec, _b_spec, _b_spec,
              pl.BlockSpec((1, 1), lambda i: (0, 0))],
    out_specs=pl.BlockSpec((1, 1), lambda i: (0, 0)),
    out_shape=jax.ShapeDtypeStruct((1, 1), jnp.float32),
    scratch_shapes=[pltpu.VMEM((1, D), jnp.float32)],
)


def kernel(x, edge_index, W_self1, W_neigh1, b1, W_self2, W_neigh2, b2,
           fc1_W, fc1_b, fc2_W, fc2_b):
  pad = EP - E
  srcp = jnp.concatenate([edge_index[0], jnp.zeros((pad,), jnp.int32)])
  dstp = jnp.concatenate([edge_index[1], jnp.full((pad,), N, jnp.int32)])
  srcs = jnp.stack([2 * srcp, 2 * srcp + 1]).reshape(NC, NS, CH, C)
  dsts = dstp.reshape(NS, CH, C)

  p1, deg = _sc_agg_deg(x.astype(jnp.bfloat16).reshape(2 * N, DH), srcs, dsts)
  h1b = _tc_layer(x, p1, deg, W_self1, W_neigh1, b1.reshape(1, D))
  (p2,) = _sc_agg(h1b.reshape(2 * N, DH), srcs, dsts)
  return _tc_layer_pool(h1b, p2, deg, W_self2, W_neigh2, b2.reshape(1, D),
                        fc1_W, fc1_b.reshape(1, D), fc2_W.reshape(1, D),
                        fc2_b.reshape(1, 1))


# speedup vs baseline: 1.7570x; 1.7296x over previous
import jax
import jax.numpy as jnp
from jax import lax
from jax.experimental import pallas as pl
from jax.experimental.pallas import tpu as pltpu
from jax.experimental.pallas import tpu_sc as plsc

N = 10000
E = 320000
D = 128
DH = D // 2

NC = 2
NS = 16
C = 128
EP = 327680
CH = EP // (NS * C)
ACC_ROWS = 10240
ZR = 64
DW = 16
ROWS_Z = ACC_ROWS // NS
K_DEG = 8
K_NODEG = 8

_mesh = plsc.VectorSubcoreMesh(
    core_axis_name="c", subcore_axis_name="s", num_cores=NC, num_subcores=NS)


def _fill(ref, rows, width, value):
  lanes = 32 if ref.dtype == jnp.bfloat16 else 16
  v = jnp.full((lanes,), value, ref.dtype)
  for r in range(rows):
    for j in range(width // lanes):
      ref[r, pl.ds(j * lanes, lanes)] = v


def _sc_agg_build(with_deg):
  K = K_DEG if with_deg else K_NODEG

  def body(tbl_hbm, srcs_hbm, dsts_hbm, *rest):
    if with_deg:
      (out_hbm, deg_hbm, src_idx, dst_idx, rows, zrow, dzrow, ones, acc,
       dacc, tbl_s, sem, ssem) = rest
    else:
      (out_hbm, src_idx, dst_idx, rows, zrow, acc, tbl_s, sem, ssem) = rest
    cid = lax.axis_index("c")
    sid = lax.axis_index("s")

    _fill(zrow, ZR, DH, 0.0)
    if with_deg:
      _fill(dzrow, ZR, DW, 0.0)
      _fill(ones, C, DW, 1.0)

    zbase = sid * ROWS_Z
    cps = []
    for k in range(ROWS_Z // ZR):
      cps.append(pltpu.async_copy(zrow, acc.at[pl.ds(zbase + k * ZR, ZR)], sem))
      if with_deg:
        cps.append(
            pltpu.async_copy(dzrow, dacc.at[pl.ds(zbase + k * ZR, ZR)], sem))
    tb = sid * (N // NS)
    cps.append(pltpu.async_copy(tbl_hbm.at[cid, pl.ds(tb, N // NS)],
                                tbl_s.at[pl.ds(tb, N // NS)], sem))
    pltpu.sync_copy(srcs_hbm.at[sid], src_idx)
    pltpu.sync_copy(dsts_hbm.at[sid], dst_idx)
    for cp in cps:
      cp.wait()
    plsc.subcore_barrier()

    def prime(b, carry):
      pltpu.async_copy(tbl_s.at[src_idx.at[b]], rows.at[b], sem)
      return carry

    lax.fori_loop(0, K, prime, 0)

    def chunk(i, carry):
      b = lax.rem(i, K)
      pltpu.make_async_copy(tbl_s.at[src_idx.at[i]], rows.at[b],
                            sem).wait()
      pltpu.async_copy(rows.at[b], acc.at[dst_idx.at[i]], ssem, add=True)
      if with_deg:
        @pl.when(cid == 0)
        def _():
          pltpu.async_copy(ones, dacc.at[dst_idx.at[i]], ssem, add=True)
      pltpu.make_async_copy(rows.at[b], acc.at[dst_idx.at[i]], ssem).wait()
      if with_deg:
        @pl.when(cid == 0)
        def _():
          pltpu.make_async_copy(ones, dacc.at[dst_idx.at[i]], ssem).wait()
      nxt = i + K

      @pl.when(nxt < CH)
      def _():
        pltpu.async_copy(tbl_s.at[src_idx.at[nxt]], rows.at[b], sem)
      return carry

    lax.fori_loop(0, CH, chunk, 0)
    plsc.subcore_barrier()

    pltpu.sync_copy(acc.at[pl.ds(zbase, ROWS_Z)],
                    out_hbm.at[cid, pl.ds(zbase, ROWS_Z)])
    if with_deg:
      @pl.when(cid == 0)
      def _():
        pltpu.sync_copy(dacc.at[pl.ds(zbase, ROWS_Z)],
                        deg_hbm.at[pl.ds(zbase, ROWS_Z)])

  out_type = [jax.ShapeDtypeStruct((NC, ACC_ROWS, DH), jnp.bfloat16)]
  scratch = [
      pltpu.VMEM((CH, C), jnp.int32),
      pltpu.VMEM((CH, C), jnp.int32),
      pltpu.VMEM((K, C, DH), jnp.bfloat16),
      pltpu.VMEM((ZR, DH), jnp.bfloat16),
  ]
  if with_deg:
    out_type.append(jax.ShapeDtypeStruct((ACC_ROWS, DW), jnp.float32))
    scratch += [
        pltpu.VMEM((ZR, DW), jnp.float32),
        pltpu.VMEM((C, DW), jnp.float32),
    ]
  scratch.append(pltpu.VMEM_SHARED((ACC_ROWS, DH), jnp.bfloat16))
  if with_deg:
    scratch.append(pltpu.VMEM_SHARED((ACC_ROWS, DW), jnp.float32))
  scratch.append(pltpu.VMEM_SHARED((N, DH), jnp.bfloat16))
  scratch.append(pltpu.SemaphoreType.DMA)
  scratch.append(pltpu.SemaphoreType.DMA)

  return pl.kernel(body, out_type=tuple(out_type), mesh=_mesh,
                   scratch_types=tuple(scratch),
                   compiler_params=pltpu.CompilerParams(
                       use_tc_tiling_on_sc=False))


_sc_agg_deg = _sc_agg_build(True)
_sc_agg = _sc_agg_build(False)

BR = 1000
GRID = N // BR


def _neigh(p, dp):
  n = jnp.concatenate([p[0], p[1]], axis=1).astype(jnp.float32)
  scale = 1.0 / jnp.maximum(dp[:, 0:1], 1.0)
  return n * scale


def _layer(h, n, ws, wn, b):
  return jnp.maximum(
      jnp.dot(h, ws, preferred_element_type=jnp.float32)
      + jnp.dot(n, wn, preferred_element_type=jnp.float32) + b, 0.0)


def _tc_layer_body(x_ref, p_ref, dp_ref, ws_ref, wn_ref, b_ref, ob_ref):
  h1 = _layer(x_ref[...], _neigh(p_ref[...], dp_ref[...]), ws_ref[...],
              wn_ref[...], b_ref[...]).astype(jnp.bfloat16)
  ob_ref[0] = h1[:, :DH]
  ob_ref[1] = h1[:, DH:]


def _tc_layer_pool_body(h_ref, p_ref, dp_ref, ws_ref, wn_ref, b_ref,
                        w1_ref, b1_ref, w2t_ref, b2_ref, o_ref, acc_ref):
  i = pl.program_id(0)
  hb = h_ref[...]
  h = jnp.concatenate([hb[0], hb[1]], axis=1).astype(jnp.float32)
  h2 = _layer(h, _neigh(p_ref[...], dp_ref[...]), ws_ref[...], wn_ref[...],
              b_ref[...])

  @pl.when(i == 0)
  def _():
    acc_ref[...] = jnp.zeros_like(acc_ref)

  acc_ref[...] += jnp.sum(h2, axis=0, keepdims=True)

  @pl.when(i == GRID - 1)
  def _():
    hg = acc_ref[...] * (1.0 / N)
    a = jnp.maximum(
        jnp.dot(hg, w1_ref[...], preferred_element_type=jnp.float32)
        + b1_ref[...], 0.0)
    o = jnp.sum(a * w2t_ref[...], axis=1, keepdims=True) + b2_ref[...]
    o_ref[...] = jax.nn.sigmoid(o)


_row_spec = pl.BlockSpec((BR, D), lambda i: (i, 0))
_split_spec = pl.BlockSpec((NC, BR, DH), lambda i: (0, i, 0))
_dp_spec = pl.BlockSpec((BR, DW), lambda i: (i, 0))
_w_spec = pl.BlockSpec((D, D), lambda i: (0, 0))
_b_spec = pl.BlockSpec((1, D), lambda i: (0, 0))

_tc_layer = pl.pallas_call(
    _tc_layer_body,
    grid=(GRID,),
    in_specs=[_row_spec, _split_spec, _dp_spec, _w_spec, _w_spec, _b_spec],
    out_specs=pl.BlockSpec((NC, BR, DH), lambda i: (0, i, 0)),
    out_shape=jax.ShapeDtypeStruct((NC, N, DH), jnp.bfloat16),
)

_tc_layer_pool = pl.pallas_call(
    _tc_layer_pool_body,
    grid=(GRID,),
    in_specs=[_split_spec, _split_spec, _dp_spec, _w_spec, _w_spec, _b_spec,
              _w_spec, _b_spec, _b_spec,
              pl.BlockSpec((1, 1), lambda i: (0, 0))],
    out_specs=pl.BlockSpec((1, 1), lambda i: (0, 0)),
    out_shape=jax.ShapeDtypeStruct((1, 1), jnp.float32),
    scratch_shapes=[pltpu.VMEM((1, D), jnp.float32)],
)


def kernel(x, edge_index, W_self1, W_neigh1, b1, W_self2, W_neigh2, b2,
           fc1_W, fc1_b, fc2_W, fc2_b):
  pad = EP - E
  srcp = jnp.concatenate([edge_index[0], jnp.zeros((pad,), jnp.int32)])
  dstp = jnp.concatenate([edge_index[1], jnp.full((pad,), N, jnp.int32)])
  srcs = srcp.reshape(NS, CH, C)
  dsts = dstp.reshape(NS, CH, C)

  xb = x.astype(jnp.bfloat16)
  x2 = jnp.stack([xb[:, :DH], xb[:, DH:]])
  p1, deg = _sc_agg_deg(x2, srcs, dsts)
  h1b = _tc_layer(x, p1, deg, W_self1, W_neigh1, b1.reshape(1, D))
  (p2,) = _sc_agg(h1b, srcs, dsts)
  return _tc_layer_pool(h1b, p2, deg, W_self2, W_neigh2, b2.reshape(1, D),
                        fc1_W, fc1_b.reshape(1, D), fc2_W.reshape(1, D),
                        fc2_b.reshape(1, 1))


# speedup vs baseline: 1.7858x; 1.0164x over previous
import jax
import jax.numpy as jnp
from jax import lax
from jax.experimental import pallas as pl
from jax.experimental.pallas import tpu as pltpu
from jax.experimental.pallas import tpu_sc as plsc

N = 10000
E = 320000
D = 128
DH = D // 2

NC = 2
NS = 16
C = 128
EP = 327680
CH = EP // (NS * C)
ACC_ROWS = 10240
ZR = 64
DW = 16
ROWS_Z = ACC_ROWS // NS
K_DEG = 8
K_NODEG = 10

_mesh = plsc.VectorSubcoreMesh(
    core_axis_name="c", subcore_axis_name="s", num_cores=NC, num_subcores=NS)


def _fill(ref, rows, width, value):
  lanes = 32 if ref.dtype == jnp.bfloat16 else 16
  v = jnp.full((lanes,), value, ref.dtype)
  for r in range(rows):
    for j in range(width // lanes):
      ref[r, pl.ds(j * lanes, lanes)] = v


def _sc_agg_build(with_deg):
  K = K_DEG if with_deg else K_NODEG

  def body(tbl_hbm, srcs_hbm, dsts_hbm, *rest):
    if with_deg:
      (out_hbm, deg_hbm, src_idx, dst_idx, rows, zrow, dzrow, ones, acc,
       dacc, tbl_s, sem, ssem) = rest
    else:
      (out_hbm, src_idx, dst_idx, rows, zrow, acc, tbl_s, sem, ssem) = rest
    cid = lax.axis_index("c")
    sid = lax.axis_index("s")

    _fill(zrow, ZR, DH, 0.0)
    if with_deg:
      _fill(dzrow, ZR, DW, 0.0)
      _fill(ones, C, DW, 1.0)

    zbase = sid * ROWS_Z
    cps = []
    for k in range(ROWS_Z // ZR):
      cps.append(pltpu.async_copy(zrow, acc.at[pl.ds(zbase + k * ZR, ZR)], sem))
      if with_deg:
        cps.append(
            pltpu.async_copy(dzrow, dacc.at[pl.ds(zbase + k * ZR, ZR)], sem))
    tb = sid * (N // NS)
    cps.append(pltpu.async_copy(tbl_hbm.at[cid, pl.ds(tb, N // NS)],
                                tbl_s.at[pl.ds(tb, N // NS)], sem))
    pltpu.sync_copy(srcs_hbm.at[sid], src_idx)
    pltpu.sync_copy(dsts_hbm.at[sid], dst_idx)
    for cp in cps:
      cp.wait()
    plsc.subcore_barrier()

    def prime(b, carry):
      pltpu.async_copy(tbl_s.at[src_idx.at[b]], rows.at[b], sem)
      return carry

    lax.fori_loop(0, K, prime, 0)

    def chunk(i, carry):
      b = lax.rem(i, K)
      pltpu.make_async_copy(tbl_s.at[src_idx.at[i]], rows.at[b],
                            sem).wait()
      pltpu.async_copy(rows.at[b], acc.at[dst_idx.at[i]], ssem, add=True)
      if with_deg:
        @pl.when(cid == 0)
        def _():
          pltpu.async_copy(ones, dacc.at[dst_idx.at[i]], ssem, add=True)
      pltpu.make_async_copy(rows.at[b], acc.at[dst_idx.at[i]], ssem).wait()
      if with_deg:
        @pl.when(cid == 0)
        def _():
          pltpu.make_async_copy(ones, dacc.at[dst_idx.at[i]], ssem).wait()
      nxt = i + K

      @pl.when(nxt < CH)
      def _():
        pltpu.async_copy(tbl_s.at[src_idx.at[nxt]], rows.at[b], sem)
      return carry

    lax.fori_loop(0, CH, chunk, 0)
    plsc.subcore_barrier()

    pltpu.sync_copy(acc.at[pl.ds(zbase, ROWS_Z)],
                    out_hbm.at[cid, pl.ds(zbase, ROWS_Z)])
    if with_deg:
      @pl.when(cid == 0)
      def _():
        pltpu.sync_copy(dacc.at[pl.ds(zbase, ROWS_Z)],
                        deg_hbm.at[pl.ds(zbase, ROWS_Z)])

  out_type = [jax.ShapeDtypeStruct((NC, ACC_ROWS, DH), jnp.bfloat16)]
  scratch = [
      pltpu.VMEM((CH, C), jnp.int32),
      pltpu.VMEM((CH, C), jnp.int32),
      pltpu.VMEM((K, C, DH), jnp.bfloat16),
      pltpu.VMEM((ZR, DH), jnp.bfloat16),
  ]
  if with_deg:
    out_type.append(jax.ShapeDtypeStruct((ACC_ROWS, DW), jnp.float32))
    scratch += [
        pltpu.VMEM((ZR, DW), jnp.float32),
        pltpu.VMEM((C, DW), jnp.float32),
    ]
  scratch.append(pltpu.VMEM_SHARED((ACC_ROWS, DH), jnp.bfloat16))
  if with_deg:
    scratch.append(pltpu.VMEM_SHARED((ACC_ROWS, DW), jnp.float32))
  scratch.append(pltpu.VMEM_SHARED((N, DH), jnp.bfloat16))
  scratch.append(pltpu.SemaphoreType.DMA)
  scratch.append(pltpu.SemaphoreType.DMA)

  return pl.kernel(body, out_type=tuple(out_type), mesh=_mesh,
                   scratch_types=tuple(scratch),
                   compiler_params=pltpu.CompilerParams(
                       use_tc_tiling_on_sc=False))


_sc_agg_deg = _sc_agg_build(True)
_sc_agg = _sc_agg_build(False)

BR = 2000
GRID = N // BR


def _neigh(p, dp):
  n = jnp.concatenate([p[0], p[1]], axis=1).astype(jnp.float32)
  scale = 1.0 / jnp.maximum(dp[:, 0:1], 1.0)
  return n * scale


def _layer(h, n, ws, wn, b):
  return jnp.maximum(
      jnp.dot(h, ws, preferred_element_type=jnp.float32)
      + jnp.dot(n, wn, preferred_element_type=jnp.float32) + b, 0.0)


def _tc_layer_body(x_ref, p_ref, dp_ref, ws_ref, wn_ref, b_ref, ob_ref):
  h1 = _layer(x_ref[...], _neigh(p_ref[...], dp_ref[...]), ws_ref[...],
              wn_ref[...], b_ref[...]).astype(jnp.bfloat16)
  ob_ref[0] = h1[:, :DH]
  ob_ref[1] = h1[:, DH:]


def _tc_layer_pool_body(h_ref, p_ref, dp_ref, ws_ref, wn_ref, b_ref,
                        w1_ref, b1_ref, w2t_ref, b2_ref, o_ref, acc_ref):
  i = pl.program_id(0)
  hb = h_ref[...]
  h = jnp.concatenate([hb[0], hb[1]], axis=1).astype(jnp.float32)
  h2 = _layer(h, _neigh(p_ref[...], dp_ref[...]), ws_ref[...], wn_ref[...],
              b_ref[...])

  @pl.when(i == 0)
  def _():
    acc_ref[...] = jnp.zeros_like(acc_ref)

  acc_ref[...] += jnp.sum(h2, axis=0, keepdims=True)

  @pl.when(i == GRID - 1)
  def _():
    hg = acc_ref[...] * (1.0 / N)
    a = jnp.maximum(
        jnp.dot(hg, w1_ref[...], preferred_element_type=jnp.float32)
        + b1_ref[...], 0.0)
    o = jnp.sum(a * w2t_ref[...], axis=1, keepdims=True) + b2_ref[...]
    o_ref[...] = jax.nn.sigmoid(o)


_row_spec = pl.BlockSpec((BR, D), lambda i: (i, 0))
_split_spec = pl.BlockSpec((NC, BR, DH), lambda i: (0, i, 0))
_dp_spec = pl.BlockSpec((BR, DW), lambda i: (i, 0))
_w_spec = pl.BlockSpec((D, D), lambda i: (0, 0))
_b_spec = pl.BlockSpec((1, D), lambda i: (0, 0))

_tc_layer = pl.pallas_call(
    _tc_layer_body,
    grid=(GRID,),
    in_specs=[_row_spec, _split_spec, _dp_spec, _w_spec, _w_spec, _b_spec],
    out_specs=pl.BlockSpec((NC, BR, DH), lambda i: (0, i, 0)),
    out_shape=jax.ShapeDtypeStruct((NC, N, DH), jnp.bfloat16),
)

_tc_layer_pool = pl.pallas_call(
    _tc_layer_pool_body,
    grid=(GRID,),
    in_specs=[_split_spec, _split_spec, _dp_spec, _w_spec, _w_spec, _b_spec,
              _w_spec, _b_spec, _b_spec,
              pl.BlockSpec((1, 1), lambda i: (0, 0))],
    out_specs=pl.BlockSpec((1, 1), lambda i: (0, 0)),
    out_shape=jax.ShapeDtypeStruct((1, 1), jnp.float32),
    scratch_shapes=[pltpu.VMEM((1, D), jnp.float32)],
)


def kernel(x, edge_index, W_self1, W_neigh1, b1, W_self2, W_neigh2, b2,
           fc1_W, fc1_b, fc2_W, fc2_b):
  pad = EP - E
  srcp = jnp.concatenate([edge_index[0], jnp.zeros((pad,), jnp.int32)])
  dstp = jnp.concatenate([edge_index[1], jnp.full((pad,), N, jnp.int32)])
  srcs = srcp.reshape(NS, CH, C)
  dsts = dstp.reshape(NS, CH, C)

  xb = x.astype(jnp.bfloat16)
  x2 = jnp.stack([xb[:, :DH], xb[:, DH:]])
  p1, deg = _sc_agg_deg(x2, srcs, dsts)
  h1b = _tc_layer(x, p1, deg, W_self1, W_neigh1, b1.reshape(1, D))
  (p2,) = _sc_agg(h1b, srcs, dsts)
  return _tc_layer_pool(h1b, p2, deg, W_self2, W_neigh2, b2.reshape(1, D),
                        fc1_W, fc1_b.reshape(1, D), fc2_W.reshape(1, D),
                        fc2_b.reshape(1, 1))


# speedup vs baseline: 1.9206x; 1.0755x over previous
import jax
import jax.numpy as jnp
from jax import lax
from jax.experimental import pallas as pl
from jax.experimental.pallas import tpu as pltpu
from jax.experimental.pallas import tpu_sc as plsc

N = 10000
E = 320000
D = 128
DH = D // 2

NC = 2
NS = 16
C = 128
TPC = 20480
TPC_LAST = E - 15 * TPC
CH = TPC // C
CH_LAST = TPC_LAST // C
ACC_ROWS = 10240
ZR = 64
DW = 16
ROWS_Z = ACC_ROWS // NS
K_DEG = 8
K_NODEG = 10

_mesh = plsc.VectorSubcoreMesh(
    core_axis_name="c", subcore_axis_name="s", num_cores=NC, num_subcores=NS)


def _fill(ref, rows, width, value):
  lanes = 32 if ref.dtype == jnp.bfloat16 else 16
  v = jnp.full((lanes,), value, ref.dtype)
  for r in range(rows):
    for j in range(width // lanes):
      ref[r, pl.ds(j * lanes, lanes)] = v


def _sc_agg_build(with_deg):
  K = K_DEG if with_deg else K_NODEG

  def body(tbl_hbm, ei_hbm, *rest):
    if with_deg:
      (out_hbm, deg_hbm, src_idx, dst_idx, rows, zrow, dzrow, ones, acc,
       dacc, tbl_s, sem, ssem) = rest
    else:
      (out_hbm, src_idx, dst_idx, rows, zrow, acc, tbl_s, sem, ssem) = rest
    cid = lax.axis_index("c")
    sid = lax.axis_index("s")

    _fill(zrow, ZR, DH, 0.0)
    if with_deg:
      _fill(dzrow, ZR, DW, 0.0)
      _fill(ones, C, DW, 1.0)

    zbase = sid * ROWS_Z
    cps = []
    for k in range(ROWS_Z // ZR):
      cps.append(pltpu.async_copy(zrow, acc.at[pl.ds(zbase + k * ZR, ZR)], sem))
      if with_deg:
        cps.append(
            pltpu.async_copy(dzrow, dacc.at[pl.ds(zbase + k * ZR, ZR)], sem))
    tb = sid * (N // NS)
    cps.append(pltpu.async_copy(tbl_hbm.at[cid, pl.ds(tb, N // NS)],
                                tbl_s.at[pl.ds(tb, N // NS)], sem))
    base = sid * TPC

    @pl.when(sid < NS - 1)
    def _():
      pltpu.sync_copy(ei_hbm.at[0, pl.ds(base, TPC)], src_idx)
      pltpu.sync_copy(ei_hbm.at[1, pl.ds(base, TPC)], dst_idx)

    @pl.when(sid == NS - 1)
    def _():
      pltpu.sync_copy(ei_hbm.at[0, pl.ds(base, TPC_LAST)],
                      src_idx.at[pl.ds(0, TPC_LAST)])
      pltpu.sync_copy(ei_hbm.at[1, pl.ds(base, TPC_LAST)],
                      dst_idx.at[pl.ds(0, TPC_LAST)])
    nch = jnp.where(sid == NS - 1, CH_LAST, CH)
    for cp in cps:
      cp.wait()
    plsc.subcore_barrier()

    def prime(b, carry):
      pltpu.async_copy(tbl_s.at[src_idx.at[pl.ds(b * C, C)]], rows.at[b], sem)
      return carry

    lax.fori_loop(0, K, prime, 0)

    def chunk(i, carry):
      b = lax.rem(i, K)
      pltpu.make_async_copy(tbl_s.at[src_idx.at[pl.ds(i * C, C)]], rows.at[b],
                            sem).wait()
      pltpu.async_copy(rows.at[b], acc.at[dst_idx.at[pl.ds(i * C, C)]], ssem, add=True)
      if with_deg:
        @pl.when(cid == 0)
        def _():
          pltpu.async_copy(ones, dacc.at[dst_idx.at[pl.ds(i * C, C)]], ssem, add=True)
      pltpu.make_async_copy(rows.at[b], acc.at[dst_idx.at[pl.ds(i * C, C)]], ssem).wait()
      if with_deg:
        @pl.when(cid == 0)
        def _():
          pltpu.make_async_copy(ones, dacc.at[dst_idx.at[pl.ds(i * C, C)]], ssem).wait()
      nxt = i + K

      @pl.when(nxt < nch)
      def _():
        pltpu.async_copy(tbl_s.at[src_idx.at[pl.ds(nxt * C, C)]], rows.at[b], sem)
      return carry

    lax.fori_loop(0, nch, chunk, 0)
    plsc.subcore_barrier()

    pltpu.sync_copy(acc.at[pl.ds(zbase, ROWS_Z)],
                    out_hbm.at[cid, pl.ds(zbase, ROWS_Z)])
    if with_deg:
      @pl.when(cid == 0)
      def _():
        pltpu.sync_copy(dacc.at[pl.ds(zbase, ROWS_Z)],
                        deg_hbm.at[pl.ds(zbase, ROWS_Z)])

  out_type = [jax.ShapeDtypeStruct((NC, ACC_ROWS, DH), jnp.bfloat16)]
  scratch = [
      pltpu.VMEM((TPC,), jnp.int32),
      pltpu.VMEM((TPC,), jnp.int32),
      pltpu.VMEM((K, C, DH), jnp.bfloat16),
      pltpu.VMEM((ZR, DH), jnp.bfloat16),
  ]
  if with_deg:
    out_type.append(jax.ShapeDtypeStruct((ACC_ROWS, DW), jnp.float32))
    scratch += [
        pltpu.VMEM((ZR, DW), jnp.float32),
        pltpu.VMEM((C, DW), jnp.float32),
    ]
  scratch.append(pltpu.VMEM_SHARED((ACC_ROWS, DH), jnp.bfloat16))
  if with_deg:
    scratch.append(pltpu.VMEM_SHARED((ACC_ROWS, DW), jnp.float32))
  scratch.append(pltpu.VMEM_SHARED((N, DH), jnp.bfloat16))
  scratch.append(pltpu.SemaphoreType.DMA)
  scratch.append(pltpu.SemaphoreType.DMA)

  return pl.kernel(body, out_type=tuple(out_type), mesh=_mesh,
                   scratch_types=tuple(scratch),
                   compiler_params=pltpu.CompilerParams(
                       use_tc_tiling_on_sc=False))


_sc_agg_deg = _sc_agg_build(True)
_sc_agg = _sc_agg_build(False)

BR = 2000
GRID = N // BR


def _neigh(p, dp):
  n = jnp.concatenate([p[0], p[1]], axis=1).astype(jnp.float32)
  scale = 1.0 / jnp.maximum(dp[:, 0:1], 1.0)
  return n * scale


def _layer(h, n, ws, wn, b):
  return jnp.maximum(
      jnp.dot(h, ws, preferred_element_type=jnp.float32)
      + jnp.dot(n, wn, preferred_element_type=jnp.float32) + b, 0.0)


def _tc_layer_body(x_ref, p_ref, dp_ref, ws_ref, wn_ref, b_ref, ob_ref):
  h1 = _layer(x_ref[...], _neigh(p_ref[...], dp_ref[...]), ws_ref[...],
              wn_ref[...], b_ref[...]).astype(jnp.bfloat16)
  ob_ref[0] = h1[:, :DH]
  ob_ref[1] = h1[:, DH:]


def _tc_layer_pool_body(h_ref, p_ref, dp_ref, ws_ref, wn_ref, b_ref,
                        w1_ref, b1_ref, w2t_ref, b2_ref, o_ref, acc_ref):
  i = pl.program_id(0)
  hb = h_ref[...]
  h = jnp.concatenate([hb[0], hb[1]], axis=1).astype(jnp.float32)
  h2 = _layer(h, _neigh(p_ref[...], dp_ref[...]), ws_ref[...], wn_ref[...],
              b_ref[...])

  @pl.when(i == 0)
  def _():
    acc_ref[...] = jnp.zeros_like(acc_ref)

  acc_ref[...] += jnp.sum(h2, axis=0, keepdims=True)

  @pl.when(i == GRID - 1)
  def _():
    hg = acc_ref[...] * (1.0 / N)
    a = jnp.maximum(
        jnp.dot(hg, w1_ref[...], preferred_element_type=jnp.float32)
        + b1_ref[...], 0.0)
    o = jnp.sum(a * w2t_ref[...], axis=1, keepdims=True) + b2_ref[...]
    o_ref[...] = jax.nn.sigmoid(o)


_row_spec = pl.BlockSpec((BR, D), lambda i: (i, 0))
_split_spec = pl.BlockSpec((NC, BR, DH), lambda i: (0, i, 0))
_dp_spec = pl.BlockSpec((BR, DW), lambda i: (i, 0))
_w_spec = pl.BlockSpec((D, D), lambda i: (0, 0))
_b_spec = pl.BlockSpec((1, D), lambda i: (0, 0))

_tc_layer = pl.pallas_call(
    _tc_layer_body,
    grid=(GRID,),
    in_specs=[_row_spec, _split_spec, _dp_spec, _w_spec, _w_spec, _b_spec],
    out_specs=pl.BlockSpec((NC, BR, DH), lambda i: (0, i, 0)),
    out_shape=jax.ShapeDtypeStruct((NC, N, DH), jnp.bfloat16),
)

_tc_layer_pool = pl.pallas_call(
    _tc_layer_pool_body,
    grid=(GRID,),
    in_specs=[_split_spec, _split_spec, _dp_spec, _w_spec, _w_spec, _b_spec,
              _w_spec, _b_spec, _b_spec,
              pl.BlockSpec((1, 1), lambda i: (0, 0))],
    out_specs=pl.BlockSpec((1, 1), lambda i: (0, 0)),
    out_shape=jax.ShapeDtypeStruct((1, 1), jnp.float32),
    scratch_shapes=[pltpu.VMEM((1, D), jnp.float32)],
)


def kernel(x, edge_index, W_self1, W_neigh1, b1, W_self2, W_neigh2, b2,
           fc1_W, fc1_b, fc2_W, fc2_b):
  xb = x.astype(jnp.bfloat16)
  x2 = jnp.stack([xb[:, :DH], xb[:, DH:]])
  p1, deg = _sc_agg_deg(x2, edge_index)
  h1b = _tc_layer(x, p1, deg, W_self1, W_neigh1, b1.reshape(1, D))
  (p2,) = _sc_agg(h1b, edge_index)
  return _tc_layer_pool(h1b, p2, deg, W_self2, W_neigh2, b2.reshape(1, D),
                        fc1_W, fc1_b.reshape(1, D), fc2_W.reshape(1, D),
                        fc2_b.reshape(1, 1))
